# Initial kernel scaffold; baseline (speedup 1.0000x reference)
import jax
import jax.numpy as jnp
from jax.experimental import pallas as pl


def kernel(x, edge_index, spatial_coords, q_w, k_w, v_w, d_w1, d_b1, d_w2, d_b2, dist_emb, dir_emb, sp_w1, sp_b1, sp_w2, sp_b2, temperature, out_w, out_b):
    raise NotImplementedError("write your pallas kernel here")


# speedup vs baseline: 1.0711x; 1.0711x over previous
import math

import jax
import jax.numpy as jnp
from jax.experimental import pallas as pl

N = 10000
HID = 128
HEADS = 8
DH = HID // HEADS
MAXD = 500.0
RADIUS = 50.0


def _ident_tc(a):
    def body(a_ref, o_ref):
        o_ref[...] = a_ref[...]
    return pl.pallas_call(
        body, out_shape=jax.ShapeDtypeStruct(a.shape, a.dtype))(a)


def _spatial_density(coords, radius=RADIUS, chunk=1000):
    NP = 10240
    R = 1024
    C = 2048
    xs = jnp.pad(coords[:, 0], (0, NP - N), constant_values=1e9)
    ys = jnp.pad(coords[:, 1], (0, NP - N), constant_values=1e9)

    xsr = xs.reshape(NP, 1)
    ysr = ys.reshape(NP, 1)
    xsc = xs.reshape(1, NP)
    ysc = ys.reshape(1, NP)

    def body(xc_ref, yc_ref, xr_ref, yr_ref, out_ref):
        cx = xc_ref[...]
        cy = yc_ref[...]

        def step(j, acc):
            off = pl.multiple_of(j * C, 8)
            ox = xr_ref[pl.ds(off, C), :]
            oy = yr_ref[pl.ds(off, C), :]
            dx = cx - ox
            dy = cy - oy
            d = jnp.sqrt(dx * dx + dy * dy + 1e-12)
            return acc + jnp.sum(jnp.where(d <= RADIUS, 1.0, 0.0), axis=0, keepdims=True)

        acc = jax.lax.fori_loop(0, NP // C, step, jnp.zeros((1, R), jnp.float32))
        out_ref[...] = acc - 1.0

    out = pl.pallas_call(
        body,
        out_shape=jax.ShapeDtypeStruct((1, NP), jnp.float32),
        grid=(NP // R,),
        in_specs=[
            pl.BlockSpec((1, R), lambda i: (0, i)),
            pl.BlockSpec((1, R), lambda i: (0, i)),
            pl.BlockSpec((NP, 1), lambda i: (0, 0)),
            pl.BlockSpec((NP, 1), lambda i: (0, 0)),
        ],
        out_specs=pl.BlockSpec((1, R), lambda i: (0, i)),
    )(xsc, ysc, xsr, ysr)
    dens = out.reshape(NP)[:N]
    return dens / (dens.max() + 1e-08)


def kernel(x, edge_index, spatial_coords, q_w, k_w, v_w, d_w1, d_b1, d_w2, d_b2,
           dist_emb, dir_emb, sp_w1, sp_b1, sp_w2, sp_b2, temperature, out_w, out_b):
    n = x.shape[0]
    row, col = edge_index[0], edge_index[1]
    deg = jnp.zeros((n,), dtype=jnp.float32).at[row].add(1.0)
    dens = _spatial_density(spatial_coords)
    nf = jnp.zeros_like(x).at[row].add(x[col])
    cnt = jnp.clip(deg, 1.0, None)[:, None]
    nmean = nf / cnt
    fv = jnp.linalg.norm(x - nmean, axis=-1)
    fv = fv / (fv.max() + 1e-08)
    df = jnp.stack([deg / (deg.max() + 1e-08), dens, fv], axis=-1)
    density_features = jax.nn.relu(df @ d_w1 + d_b1) @ d_w2 + d_b2
    q = (x @ q_w).reshape(n, HEADS, DH)
    k = (x @ k_w).reshape(n, HEADS, DH)
    v = (x @ v_w).reshape(n, HEADS, DH)
    rel = spatial_coords[col] - spatial_coords[row]
    dist = jnp.linalg.norm(rel, axis=-1)
    ang = jnp.arctan2(rel[:, 1], rel[:, 0])
    dbin = jnp.clip((dist / MAXD * 99.0).astype(jnp.int32), 0, 99)
    abin = jnp.clip(((ang + math.pi) / (2.0 * math.pi) * 15.0).astype(jnp.int32), 0, 15)
    semb = jnp.concatenate([dist_emb[dbin], dir_emb[abin]], axis=-1)
    sp = (jax.nn.relu(semb @ sp_w1 + sp_b1) @ sp_w2 + sp_b2)[:, 0]
    qmax = jnp.sqrt((q * q).sum(-1)).max(0)
    kmax = jnp.sqrt((k * k).sum(-1)).max(0)
    se_all = jnp.concatenate([
        jnp.repeat(dist_emb, 16, axis=0),
        jnp.tile(dir_emb, (100, 1)),
    ], axis=-1)
    sp_all = (jax.nn.relu(se_all @ sp_w1 + sp_b1) @ sp_w2 + sp_b2)[:, 0]
    spmax = jnp.abs(sp_all).max()
    shift = (qmax * kmax / math.sqrt(DH) + spmax) / jnp.abs(temperature)

    alpha = (q[row] * k[col]).sum(-1) / math.sqrt(DH)
    alpha = (alpha + sp[:, None]) / temperature[None, :]
    ex = jnp.exp(alpha - shift[None, :])
    den = jnp.zeros((n, HEADS), dtype=jnp.float32).at[row].add(ex) + 1e-16
    acc = jnp.zeros((n, HEADS, DH), dtype=jnp.float32).at[row].add(ex[:, :, None] * v[col])
    out = acc / den[:, :, None]
    out = out.reshape(n, HID) + density_features
    return _ident_tc(out @ out_w + out_b)


# speedup vs baseline: 21.6456x; 20.2091x over previous
import functools
import math

import jax
import jax.numpy as jnp
from jax import lax
from jax.experimental import pallas as pl
from jax.experimental.pallas import tpu as pltpu
from jax.experimental.pallas import tpu_sc as plsc

N = 10000
E = 320000
HID = 128
HEADS = 8
DH = HID // HEADS
MAXD = 500.0
RADIUS = 50.0

NC = 2
NS = 16
NW = NC * NS
CH = 128
NCHUNK = E // CH
CPW = (NCHUNK + NW - 1) // NW
NPAD = 10240
NPS = NPAD // NS


def _proj_tc(x, w3):
    def body(x_ref, w_ref, qkv_ref, qn_ref, kn_ref):
        xv = x_ref[...]
        qkv = jnp.dot(xv, w_ref[...], preferred_element_type=jnp.float32)
        qkv_ref[...] = qkv
        r = lax.broadcasted_iota(jnp.int32, (HID, HEADS), 0)
        c = lax.broadcasted_iota(jnp.int32, (HID, HEADS), 1)
        sel = (r // DH == c).astype(jnp.float32)
        q = qkv[:, :HID]
        k = qkv[:, HID:2 * HID]
        qn2 = jnp.dot(q * q, sel, preferred_element_type=jnp.float32)
        kn2 = jnp.dot(k * k, sel, preferred_element_type=jnp.float32)
        qn_ref[...] = jnp.max(qn2, axis=0, keepdims=True)
        kn_ref[...] = jnp.max(kn2, axis=0, keepdims=True)

    return pl.pallas_call(
        body,
        out_shape=(
            jax.ShapeDtypeStruct((N, 3 * HID), jnp.float32),
            jax.ShapeDtypeStruct((1, HEADS), jnp.float32),
            jax.ShapeDtypeStruct((1, HEADS), jnp.float32),
        ),
    )(x, w3)


def _spatial_density(coords):
    NP = 10240
    R = 1024
    C = 2048
    xs = jnp.pad(coords[:, 0], (0, NP - N), constant_values=1e9)
    ys = jnp.pad(coords[:, 1], (0, NP - N), constant_values=1e9)

    xsr = xs.reshape(NP, 1)
    ysr = ys.reshape(NP, 1)
    xsc = xs.reshape(1, NP)
    ysc = ys.reshape(1, NP)

    def body(xc_ref, yc_ref, xr_ref, yr_ref, out_ref):
        cx = xc_ref[...]
        cy = yc_ref[...]

        def step(j, acc):
            off = pl.multiple_of(j * C, 8)
            ox = xr_ref[pl.ds(off, C), :]
            oy = yr_ref[pl.ds(off, C), :]
            dx = cx - ox
            dy = cy - oy
            d = jnp.sqrt(dx * dx + dy * dy + 1e-12)
            return acc + jnp.sum(jnp.where(d <= RADIUS, 1.0, 0.0), axis=0, keepdims=True)

        acc = jax.lax.fori_loop(0, NP // C, step, jnp.zeros((1, R), jnp.float32))
        out_ref[...] = acc - 1.0

    out = pl.pallas_call(
        body,
        out_shape=jax.ShapeDtypeStruct((1, NP), jnp.float32),
        grid=(NP // R,),
        in_specs=[
            pl.BlockSpec((1, R), lambda i: (0, i)),
            pl.BlockSpec((1, R), lambda i: (0, i)),
            pl.BlockSpec((NP, 1), lambda i: (0, 0)),
            pl.BlockSpec((NP, 1), lambda i: (0, 0)),
        ],
        out_specs=pl.BlockSpec((1, R), lambda i: (0, i)),
    )(xsc, ysc, xsr, ysr)
    dens = out.reshape(NP)[:N]
    return dens / (dens.max() + 1e-08)


def _sc_gather(q, k, v, x, c128, row2d, col2d, z128):
    mesh = plsc.VectorSubcoreMesh(core_axis_name="c", subcore_axis_name="s")

    @functools.partial(
        pl.kernel,
        mesh=mesh,
        out_type=(
            jax.ShapeDtypeStruct((E, HID), jnp.float32),
            jax.ShapeDtypeStruct((E, HID), jnp.float32),
            jax.ShapeDtypeStruct((E, HID), jnp.float32),
            jax.ShapeDtypeStruct((E, HID), jnp.float32),
            jax.ShapeDtypeStruct((E, HID), jnp.float32),
            jax.ShapeDtypeStruct((NC, NPAD, HID), jnp.float32),
        ),
        scratch_types=[
            pltpu.VMEM((CH,), jnp.int32),
            pltpu.VMEM((CH,), jnp.int32),
            pltpu.VMEM((CH, HID), jnp.float32),
            pltpu.VMEM_SHARED((NPAD, HID), jnp.float32),
            pltpu.SemaphoreType.DMA,
        ],
    )
    def sck(q_hbm, k_hbm, v_hbm, x_hbm, c_hbm, row_hbm, col_hbm, z_hbm,
            qr_hbm, kc_hbm, vc_hbm, rc_hbm, cc_hbm, nf_hbm,
            idxr_v, idxc_v, buf, nf_sh, sem):
        cid = lax.axis_index("c")
        sid = lax.axis_index("s")
        wid = sid * NC + cid

        pltpu.sync_copy(z_hbm.at[pl.ds(sid * NPS, NPS)], nf_sh.at[pl.ds(sid * NPS, NPS)])
        plsc.subcore_barrier()

        def chunk_body(i, carry):
            t = wid + i * NW

            @pl.when(t < NCHUNK)
            def _():
                base = t * CH
                pltpu.sync_copy(row_hbm.at[t], idxr_v)
                pltpu.sync_copy(col_hbm.at[t], idxc_v)
                pltpu.async_copy(q_hbm.at[idxr_v], buf, sem).wait()
                pltpu.sync_copy(buf, qr_hbm.at[pl.ds(base, CH)])
                pltpu.async_copy(k_hbm.at[idxc_v], buf, sem).wait()
                pltpu.sync_copy(buf, kc_hbm.at[pl.ds(base, CH)])
                pltpu.async_copy(v_hbm.at[idxc_v], buf, sem).wait()
                pltpu.sync_copy(buf, vc_hbm.at[pl.ds(base, CH)])
                pltpu.async_copy(c_hbm.at[idxr_v], buf, sem).wait()
                pltpu.sync_copy(buf, rc_hbm.at[pl.ds(base, CH)])
                pltpu.async_copy(c_hbm.at[idxc_v], buf, sem).wait()
                pltpu.sync_copy(buf, cc_hbm.at[pl.ds(base, CH)])
                pltpu.async_copy(x_hbm.at[idxc_v], buf, sem).wait()
                pltpu.sync_copy(buf, nf_sh.at[idxr_v], add=True)

            return carry

        lax.fori_loop(0, CPW, chunk_body, 0)

        plsc.subcore_barrier()
        pltpu.sync_copy(nf_sh.at[pl.ds(sid * NPS, NPS)],
                        nf_hbm.at[cid, pl.ds(sid * NPS, NPS)])

    return sck(q, k, v, x, c128, row2d, col2d, z128)


EC = 3200


def _edge_tc(qr, kc, vc, rc, cc, de_pad, di, w1, b1, w2, b2, temp, shift):
    def body(qr_ref, kc_ref, vc_ref, rc_ref, cc_ref, de_ref, di_ref,
             w1_ref, b1_ref, w2_ref, b2_ref, t_ref, s_ref, oa_ref, ob_ref):
        rcv = rc_ref[...]
        ccv = cc_ref[...]
        dxv = ccv[:, 0:1] - rcv[:, 0:1]
        dyv = ccv[:, 1:2] - rcv[:, 1:2]
        dist = jnp.sqrt(dxv * dxv + dyv * dyv)
        ang = jnp.arctan2(dyv, dxv)
        dbin = jnp.clip((dist / MAXD * 99.0).astype(jnp.int32), 0, 99)
        abin = jnp.clip(((ang + math.pi) / (2.0 * math.pi) * 15.0).astype(jnp.int32), 0, 15)

        il = lax.broadcasted_iota(jnp.int32, (EC, HID), 1)
        ohd = ((il == dbin) & (il < 100)).astype(jnp.float32)
        demb = jnp.dot(ohd, de_ref[...], preferred_element_type=jnp.float32)
        i16 = lax.broadcasted_iota(jnp.int32, (EC, 16), 1)
        oha = (i16 == abin).astype(jnp.float32)
        aemb = jnp.dot(oha, di_ref[...], preferred_element_type=jnp.float32)
        semb = jnp.concatenate([demb, aemb], axis=1)
        h = jnp.maximum(jnp.dot(semb, w1_ref[...], preferred_element_type=jnp.float32)
                        + b1_ref[...], 0.0)
        sp = jnp.dot(h, w2_ref[...], preferred_element_type=jnp.float32) + b2_ref[...]

        r = lax.broadcasted_iota(jnp.int32, (HID, HEADS), 0)
        c = lax.broadcasted_iota(jnp.int32, (HID, HEADS), 1)
        sel = (r // DH == c).astype(jnp.float32)
        prod = qr_ref[...] * kc_ref[...]
        alpha = jnp.dot(prod, sel, preferred_element_type=jnp.float32) / math.sqrt(DH)
        alpha = (alpha + sp) / t_ref[...]
        ex = jnp.exp(alpha - s_ref[...])

        r8 = lax.broadcasted_iota(jnp.int32, (HEADS, HID), 0)
        c8 = lax.broadcasted_iota(jnp.int32, (HEADS, HID), 1)
        expand = (c8 // DH == r8).astype(jnp.float32)
        exx = jnp.dot(ex, expand, preferred_element_type=jnp.float32)
        exv = vc_ref[...] * exx
        oa_ref[...] = jnp.concatenate(
            [exv[:, :64], ex, jnp.ones((EC, 1), jnp.float32),
             jnp.zeros((EC, 55), jnp.float32)], axis=1)
        ob_ref[...] = jnp.concatenate(
            [exv[:, 64:], jnp.zeros((EC, 64), jnp.float32)], axis=1)

    grid = (E // EC,)
    return pl.pallas_call(
        body,
        out_shape=(
            jax.ShapeDtypeStruct((E, HID), jnp.float32),
            jax.ShapeDtypeStruct((E, HID), jnp.float32),
        ),
        grid=grid,
        in_specs=[
            pl.BlockSpec((EC, HID), lambda i: (i, 0)),
            pl.BlockSpec((EC, HID), lambda i: (i, 0)),
            pl.BlockSpec((EC, HID), lambda i: (i, 0)),
            pl.BlockSpec((EC, HID), lambda i: (i, 0)),
            pl.BlockSpec((EC, HID), lambda i: (i, 0)),
            pl.BlockSpec((HID, HEADS), lambda i: (0, 0)),
            pl.BlockSpec((16, HEADS), lambda i: (0, 0)),
            pl.BlockSpec((DH, DH), lambda i: (0, 0)),
            pl.BlockSpec((1, DH), lambda i: (0, 0)),
            pl.BlockSpec((DH, 1), lambda i: (0, 0)),
            pl.BlockSpec((1, 1), lambda i: (0, 0)),
            pl.BlockSpec((1, HEADS), lambda i: (0, 0)),
            pl.BlockSpec((1, HEADS), lambda i: (0, 0)),
        ],
        out_specs=(
            pl.BlockSpec((EC, HID), lambda i: (i, 0)),
            pl.BlockSpec((EC, HID), lambda i: (i, 0)),
        ),
    )(qr, kc, vc, rc, cc, de_pad, di, w1, b1, w2, b2, temp, shift)


def _sc_scatter(msg, row2d, z128):
    mesh = plsc.VectorSubcoreMesh(core_axis_name="c", subcore_axis_name="s")

    @functools.partial(
        pl.kernel,
        mesh=mesh,
        out_type=jax.ShapeDtypeStruct((NC, NPAD, HID), jnp.float32),
        scratch_types=[
            pltpu.VMEM((CH,), jnp.int32),
            pltpu.VMEM((CH, HID), jnp.float32),
            pltpu.VMEM_SHARED((NPAD, HID), jnp.float32),
            pltpu.SemaphoreType.DMA,
        ],
    )
    def sck(msg_hbm, row_hbm, z_hbm, acc_hbm, idx_v, buf, acc_sh, sem):
        cid = lax.axis_index("c")
        sid = lax.axis_index("s")
        wid = sid * NC + cid

        pltpu.sync_copy(z_hbm.at[pl.ds(sid * NPS, NPS)], acc_sh.at[pl.ds(sid * NPS, NPS)])
        plsc.subcore_barrier()

        def chunk_body(i, carry):
            t = wid + i * NW

            @pl.when(t < NCHUNK)
            def _():
                pltpu.sync_copy(row_hbm.at[t], idx_v)
                pltpu.sync_copy(msg_hbm.at[pl.ds(t * CH, CH)], buf)
                pltpu.sync_copy(buf, acc_sh.at[idx_v], add=True)

            return carry

        lax.fori_loop(0, CPW, chunk_body, 0)

        plsc.subcore_barrier()
        pltpu.sync_copy(acc_sh.at[pl.ds(sid * NPS, NPS)],
                        acc_hbm.at[cid, pl.ds(sid * NPS, NPS)])

    return sck(msg, row2d, z128)


def _final_tc(accA, accB, nf, dens, x, d_w1, d_b1, d_w2, d_b2, out_w, out_b):
    def body(a_ref, b_ref, nf_ref, dens_ref, x_ref, w1_ref, b1_ref,
             w2_ref, b2_ref, ow_ref, ob_ref, o_ref):
        av = a_ref[...]
        bv = b_ref[...]
        deg_v = av[:, 72:73]
        cnt = jnp.maximum(deg_v, 1.0)
        xv = x_ref[...]
        xd = xv - nf_ref[...] / cnt
        ones = jnp.ones((HID, 1), jnp.float32)
        fv = jnp.sqrt(jnp.dot(xd * xd, ones, preferred_element_type=jnp.float32))
        fvn = fv / (jnp.max(fv) + 1e-08)
        degn = deg_v / (jnp.max(deg_v) + 1e-08)
        df = jnp.concatenate([degn, dens_ref[...], fvn], axis=1)
        hmid = jnp.maximum(
            jnp.dot(df, w1_ref[...], preferred_element_type=jnp.float32) + b1_ref[...], 0.0)
        densf = jnp.dot(hmid, w2_ref[...], preferred_element_type=jnp.float32) + b2_ref[...]

        r8 = lax.broadcasted_iota(jnp.int32, (HEADS, HID), 0)
        c8 = lax.broadcasted_iota(jnp.int32, (HEADS, HID), 1)
        expand = (c8 // DH == r8).astype(jnp.float32)
        den = jnp.dot(av[:, 64:72], expand, preferred_element_type=jnp.float32) + 1e-16
        att = jnp.concatenate([av[:, :64], bv[:, :64]], axis=1) / den
        o = att + densf
        o_ref[...] = jnp.dot(o, ow_ref[...], preferred_element_type=jnp.float32) + ob_ref[...]

    return pl.pallas_call(
        body,
        out_shape=jax.ShapeDtypeStruct((N, HID), jnp.float32),
    )(accA, accB, nf, dens, x, d_w1, d_b1, d_w2, d_b2, out_w, out_b)


def kernel(x, edge_index, spatial_coords, q_w, k_w, v_w, d_w1, d_b1, d_w2, d_b2,
           dist_emb, dir_emb, sp_w1, sp_b1, sp_w2, sp_b2, temperature, out_w, out_b):
    row = edge_index[0]
    col = edge_index[1]
    row2d = row.reshape(NCHUNK, CH)
    col2d = col.reshape(NCHUNK, CH)
    c128 = jnp.pad(spatial_coords, ((0, 0), (0, HID - 2)))

    w3 = jnp.concatenate([q_w, k_w, v_w], axis=1)
    qkv, qn2, kn2 = _proj_tc(x, w3)
    q = qkv[:, :HID]
    k = qkv[:, HID:2 * HID]
    v = qkv[:, 2 * HID:]

    se_all = jnp.concatenate([
        jnp.repeat(dist_emb, 16, axis=0),
        jnp.tile(dir_emb, (100, 1)),
    ], axis=-1)
    sp_all = (jax.nn.relu(se_all @ sp_w1 + sp_b1) @ sp_w2 + sp_b2)[:, 0]
    spmax = jnp.abs(sp_all).max()
    shift = ((jnp.sqrt(qn2) * jnp.sqrt(kn2)) / math.sqrt(DH) + spmax) \
        / jnp.abs(temperature)[None, :]

    z128 = jnp.zeros((NPAD, HID), jnp.float32)
    qr, kc, vc, rc, cc, nf2 = _sc_gather(q, k, v, x, c128, row2d, col2d, z128)

    dens = _spatial_density(spatial_coords)

    de_pad = jnp.pad(dist_emb, ((0, 28), (0, 0)))
    msgA, msgB = _edge_tc(qr, kc, vc, rc, cc, de_pad,
                          dir_emb, sp_w1, sp_b1.reshape(1, DH), sp_w2,
                          sp_b2.reshape(1, 1), temperature.reshape(1, HEADS), shift)

    maccA = _sc_scatter(msgA, row2d, z128)
    maccB = _sc_scatter(msgB, row2d, z128)

    accA = maccA[0, :N] + maccA[1, :N]
    accB = maccB[0, :N] + maccB[1, :N]
    nf = nf2[0, :N] + nf2[1, :N]
    return _final_tc(accA, accB, nf, dens.reshape(N, 1), x,
                     d_w1, d_b1.reshape(1, HID // 2), d_w2, d_b2.reshape(1, HID),
                     out_w, out_b.reshape(1, HID))


# speedup vs baseline: 23.9714x; 1.1075x over previous
import functools
import math

import jax
import jax.numpy as jnp
from jax import lax
from jax.experimental import pallas as pl
from jax.experimental.pallas import tpu as pltpu
from jax.experimental.pallas import tpu_sc as plsc

N = 10000
E = 320000
HID = 128
HEADS = 8
DH = HID // HEADS
MAXD = 500.0
RADIUS = 50.0

NC = 2
NS = 16
NW = NC * NS
CH = 128
NCHUNK = E // CH
CPW = (NCHUNK + NW - 1) // NW
NPAD = 10240
NPS = NPAD // NS


def _proj_tc(x, w3):
    def body(x_ref, w_ref, qkv_ref, qn_ref, kn_ref):
        xv = x_ref[...]
        qkv = jnp.dot(xv, w_ref[...], preferred_element_type=jnp.float32)
        qkv_ref[...] = qkv
        r = lax.broadcasted_iota(jnp.int32, (HID, HEADS), 0)
        c = lax.broadcasted_iota(jnp.int32, (HID, HEADS), 1)
        sel = (r // DH == c).astype(jnp.float32)
        q = qkv[:, :HID]
        k = qkv[:, HID:2 * HID]
        qn2 = jnp.dot(q * q, sel, preferred_element_type=jnp.float32)
        kn2 = jnp.dot(k * k, sel, preferred_element_type=jnp.float32)
        qn_ref[...] = jnp.max(qn2, axis=0, keepdims=True)
        kn_ref[...] = jnp.max(kn2, axis=0, keepdims=True)

    return pl.pallas_call(
        body,
        out_shape=(
            jax.ShapeDtypeStruct((N, 3 * HID), jnp.float32),
            jax.ShapeDtypeStruct((1, HEADS), jnp.float32),
            jax.ShapeDtypeStruct((1, HEADS), jnp.float32),
        ),
    )(x, w3)


def _spatial_density(coords):
    NP = 10240
    R = 1024
    C = 2048
    xs = jnp.pad(coords[:, 0], (0, NP - N), constant_values=1e9)
    ys = jnp.pad(coords[:, 1], (0, NP - N), constant_values=1e9)

    xsr = xs.reshape(NP, 1)
    ysr = ys.reshape(NP, 1)
    xsc = xs.reshape(1, NP)
    ysc = ys.reshape(1, NP)

    def body(xc_ref, yc_ref, xr_ref, yr_ref, out_ref):
        cx = xc_ref[...]
        cy = yc_ref[...]

        def step(j, acc):
            off = pl.multiple_of(j * C, 8)
            ox = xr_ref[pl.ds(off, C), :]
            oy = yr_ref[pl.ds(off, C), :]
            dx = cx - ox
            dy = cy - oy
            d = jnp.sqrt(dx * dx + dy * dy + 1e-12)
            return acc + jnp.sum(jnp.where(d <= RADIUS, 1.0, 0.0), axis=0, keepdims=True)

        acc = jax.lax.fori_loop(0, NP // C, step, jnp.zeros((1, R), jnp.float32))
        out_ref[...] = acc - 1.0

    out = pl.pallas_call(
        body,
        out_shape=jax.ShapeDtypeStruct((1, NP), jnp.float32),
        grid=(NP // R,),
        in_specs=[
            pl.BlockSpec((1, R), lambda i: (0, i)),
            pl.BlockSpec((1, R), lambda i: (0, i)),
            pl.BlockSpec((NP, 1), lambda i: (0, 0)),
            pl.BlockSpec((NP, 1), lambda i: (0, 0)),
        ],
        out_specs=pl.BlockSpec((1, R), lambda i: (0, i)),
    )(xsc, ysc, xsr, ysr)
    dens = out.reshape(NP)[:N]
    return dens / (dens.max() + 1e-08)


def _sc_gather(q, k, v, c128, row2d, col2d):
    mesh = plsc.VectorSubcoreMesh(core_axis_name="c", subcore_axis_name="s")

    @functools.partial(
        pl.kernel,
        mesh=mesh,
        out_type=(
            jax.ShapeDtypeStruct((E, HID), jnp.float32),
            jax.ShapeDtypeStruct((E, HID), jnp.float32),
            jax.ShapeDtypeStruct((E, HID), jnp.float32),
            jax.ShapeDtypeStruct((E, 16), jnp.float32),
        ),
        scratch_types=[
            pltpu.VMEM((CH,), jnp.int32),
            pltpu.VMEM((CH,), jnp.int32),
            pltpu.VMEM((CH, HID), jnp.float32),
            pltpu.VMEM((CH, HID), jnp.float32),
            pltpu.VMEM((CH, HID), jnp.float32),
            pltpu.VMEM((CH, HID), jnp.float32),
            pltpu.VMEM((CH, HID), jnp.float32),
            pltpu.VMEM((CH, 16), jnp.float32),
            pltpu.SemaphoreType.DMA,
        ],
    )
    def sck(q_hbm, k_hbm, v_hbm, c_hbm, row_hbm, col_hbm,
            qr_hbm, kc_hbm, vc_hbm, dxy_hbm,
            idxr_v, idxc_v, bq, bk, bv, bcr, bcc, bdxy, sem):
        cid = lax.axis_index("c")
        sid = lax.axis_index("s")
        wid = sid * NC + cid

        def chunk_body(i, carry):
            t = wid + i * NW

            @pl.when(t < NCHUNK)
            def _():
                base = t * CH
                pltpu.sync_copy(row_hbm.at[t], idxr_v)
                pltpu.sync_copy(col_hbm.at[t], idxc_v)
                cq = pltpu.async_copy(q_hbm.at[idxr_v], bq, sem)
                ck = pltpu.async_copy(k_hbm.at[idxc_v], bk, sem)
                cv = pltpu.async_copy(v_hbm.at[idxc_v], bv, sem)
                ccr = pltpu.async_copy(c_hbm.at[idxr_v], bcr, sem)
                ccc = pltpu.async_copy(c_hbm.at[idxc_v], bcc, sem)
                cq.wait()
                ck.wait()
                cv.wait()
                ccr.wait()
                ccc.wait()
                pltpu.sync_copy(bq, qr_hbm.at[pl.ds(base, CH)])
                pltpu.sync_copy(bk, kc_hbm.at[pl.ds(base, CH)])
                pltpu.sync_copy(bv, vc_hbm.at[pl.ds(base, CH)])
                for rr in range(CH):
                    bdxy[rr, :] = bcc[rr, pl.ds(0, 16)] - bcr[rr, pl.ds(0, 16)]
                pltpu.sync_copy(bdxy, dxy_hbm.at[pl.ds(base, CH)])

            return carry

        lax.fori_loop(0, CPW, chunk_body, 0)

    return sck(q, k, v, c128, row2d, col2d)


def _sc_nf(x, row2d, col2d, z128):
    mesh = plsc.VectorSubcoreMesh(core_axis_name="c", subcore_axis_name="s")

    @functools.partial(
        pl.kernel,
        mesh=mesh,
        out_type=jax.ShapeDtypeStruct((NC, NPAD, HID), jnp.float32),
        scratch_types=[
            pltpu.VMEM((CH,), jnp.int32),
            pltpu.VMEM((CH,), jnp.int32),
            pltpu.VMEM((CH, HID), jnp.float32),
            pltpu.VMEM_SHARED((NPAD, HID), jnp.float32),
            pltpu.SemaphoreType.DMA,
        ],
    )
    def sck(x_hbm, row_hbm, col_hbm, z_hbm, nf_hbm,
            idxr_v, idxc_v, buf, nf_sh, sem):
        cid = lax.axis_index("c")
        sid = lax.axis_index("s")
        wid = sid * NC + cid

        pltpu.sync_copy(z_hbm.at[pl.ds(sid * NPS, NPS)], nf_sh.at[pl.ds(sid * NPS, NPS)])
        plsc.subcore_barrier()

        def chunk_body(i, carry):
            t = wid + i * NW

            @pl.when(t < NCHUNK)
            def _():
                pltpu.sync_copy(row_hbm.at[t], idxr_v)
                pltpu.sync_copy(col_hbm.at[t], idxc_v)
                pltpu.async_copy(x_hbm.at[idxc_v], buf, sem).wait()
                pltpu.sync_copy(buf, nf_sh.at[idxr_v], add=True)

            return carry

        lax.fori_loop(0, CPW, chunk_body, 0)

        plsc.subcore_barrier()
        pltpu.sync_copy(nf_sh.at[pl.ds(sid * NPS, NPS)],
                        nf_hbm.at[cid, pl.ds(sid * NPS, NPS)])

    return sck(x, row2d, col2d, z128)


EC = 3200


def _edge_tc(qr, kc, vc, dxy, de_pad, di, w1, b1, w2, b2, temp, shift):
    def body(qr_ref, kc_ref, vc_ref, dxy_ref, de_ref, di_ref,
             w1_ref, b1_ref, w2_ref, b2_ref, t_ref, s_ref, oa_ref, ob_ref):
        dxyv = dxy_ref[...]
        dxv = dxyv[:, 0:1]
        dyv = dxyv[:, 1:2]
        dist = jnp.sqrt(dxv * dxv + dyv * dyv)
        ang = jnp.arctan2(dyv, dxv)
        dbin = jnp.clip((dist / MAXD * 99.0).astype(jnp.int32), 0, 99)
        abin = jnp.clip(((ang + math.pi) / (2.0 * math.pi) * 15.0).astype(jnp.int32), 0, 15)

        il = lax.broadcasted_iota(jnp.int32, (EC, HID), 1)
        ohd = ((il == dbin) & (il < 100)).astype(jnp.float32)
        demb = jnp.dot(ohd, de_ref[...], preferred_element_type=jnp.float32)
        i16 = lax.broadcasted_iota(jnp.int32, (EC, 16), 1)
        oha = (i16 == abin).astype(jnp.float32)
        aemb = jnp.dot(oha, di_ref[...], preferred_element_type=jnp.float32)
        semb = jnp.concatenate([demb, aemb], axis=1)
        h = jnp.maximum(jnp.dot(semb, w1_ref[...], preferred_element_type=jnp.float32)
                        + b1_ref[...], 0.0)
        sp = jnp.dot(h, w2_ref[...], preferred_element_type=jnp.float32) + b2_ref[...]

        r = lax.broadcasted_iota(jnp.int32, (HID, HEADS), 0)
        c = lax.broadcasted_iota(jnp.int32, (HID, HEADS), 1)
        sel = (r // DH == c).astype(jnp.float32)
        prod = qr_ref[...] * kc_ref[...]
        alpha = jnp.dot(prod, sel, preferred_element_type=jnp.float32) / math.sqrt(DH)
        alpha = (alpha + sp) / t_ref[...]
        ex = jnp.exp(alpha - s_ref[...])

        r8 = lax.broadcasted_iota(jnp.int32, (HEADS, HID), 0)
        c8 = lax.broadcasted_iota(jnp.int32, (HEADS, HID), 1)
        expand = (c8 // DH == r8).astype(jnp.float32)
        exx = jnp.dot(ex, expand, preferred_element_type=jnp.float32)
        exv = vc_ref[...] * exx
        oa_ref[...] = jnp.concatenate(
            [exv[:, :64], ex, jnp.ones((EC, 1), jnp.float32),
             jnp.zeros((EC, 55), jnp.float32)], axis=1)
        ob_ref[...] = jnp.concatenate(
            [exv[:, 64:], jnp.zeros((EC, 64), jnp.float32)], axis=1)

    grid = (E // EC,)
    return pl.pallas_call(
        body,
        out_shape=(
            jax.ShapeDtypeStruct((E, HID), jnp.float32),
            jax.ShapeDtypeStruct((E, HID), jnp.float32),
        ),
        grid=grid,
        in_specs=[
            pl.BlockSpec((EC, HID), lambda i: (i, 0)),
            pl.BlockSpec((EC, HID), lambda i: (i, 0)),
            pl.BlockSpec((EC, HID), lambda i: (i, 0)),
            pl.BlockSpec((EC, 16), lambda i: (i, 0)),
            pl.BlockSpec((HID, HEADS), lambda i: (0, 0)),
            pl.BlockSpec((16, HEADS), lambda i: (0, 0)),
            pl.BlockSpec((DH, DH), lambda i: (0, 0)),
            pl.BlockSpec((1, DH), lambda i: (0, 0)),
            pl.BlockSpec((DH, 1), lambda i: (0, 0)),
            pl.BlockSpec((1, 1), lambda i: (0, 0)),
            pl.BlockSpec((1, HEADS), lambda i: (0, 0)),
            pl.BlockSpec((1, HEADS), lambda i: (0, 0)),
        ],
        out_specs=(
            pl.BlockSpec((EC, HID), lambda i: (i, 0)),
            pl.BlockSpec((EC, HID), lambda i: (i, 0)),
        ),
    )(qr, kc, vc, dxy, de_pad, di, w1, b1, w2, b2, temp, shift)


def _sc_scatter(msg, row2d, z128):
    mesh = plsc.VectorSubcoreMesh(core_axis_name="c", subcore_axis_name="s")

    @functools.partial(
        pl.kernel,
        mesh=mesh,
        out_type=jax.ShapeDtypeStruct((NC, NPAD, HID), jnp.float32),
        scratch_types=[
            pltpu.VMEM((CH,), jnp.int32),
            pltpu.VMEM((CH, HID), jnp.float32),
            pltpu.VMEM_SHARED((NPAD, HID), jnp.float32),
            pltpu.SemaphoreType.DMA,
        ],
    )
    def sck(msg_hbm, row_hbm, z_hbm, acc_hbm, idx_v, buf, acc_sh, sem):
        cid = lax.axis_index("c")
        sid = lax.axis_index("s")
        wid = sid * NC + cid

        pltpu.sync_copy(z_hbm.at[pl.ds(sid * NPS, NPS)], acc_sh.at[pl.ds(sid * NPS, NPS)])
        plsc.subcore_barrier()

        def chunk_body(i, carry):
            t = wid + i * NW

            @pl.when(t < NCHUNK)
            def _():
                pltpu.sync_copy(row_hbm.at[t], idx_v)
                pltpu.sync_copy(msg_hbm.at[pl.ds(t * CH, CH)], buf)
                pltpu.sync_copy(buf, acc_sh.at[idx_v], add=True)

            return carry

        lax.fori_loop(0, CPW, chunk_body, 0)

        plsc.subcore_barrier()
        pltpu.sync_copy(acc_sh.at[pl.ds(sid * NPS, NPS)],
                        acc_hbm.at[cid, pl.ds(sid * NPS, NPS)])

    return sck(msg, row2d, z128)


def _final_tc(accA, accB, nf, dens, x, d_w1, d_b1, d_w2, d_b2, out_w, out_b):
    def body(a_ref, b_ref, nf_ref, dens_ref, x_ref, w1_ref, b1_ref,
             w2_ref, b2_ref, ow_ref, ob_ref, o_ref):
        av = a_ref[...]
        bv = b_ref[...]
        deg_v = av[:, 72:73]
        cnt = jnp.maximum(deg_v, 1.0)
        xv = x_ref[...]
        xd = xv - nf_ref[...] / cnt
        ones = jnp.ones((HID, 1), jnp.float32)
        fv = jnp.sqrt(jnp.dot(xd * xd, ones, preferred_element_type=jnp.float32))
        fvn = fv / (jnp.max(fv) + 1e-08)
        degn = deg_v / (jnp.max(deg_v) + 1e-08)
        df = jnp.concatenate([degn, dens_ref[...], fvn], axis=1)
        hmid = jnp.maximum(
            jnp.dot(df, w1_ref[...], preferred_element_type=jnp.float32) + b1_ref[...], 0.0)
        densf = jnp.dot(hmid, w2_ref[...], preferred_element_type=jnp.float32) + b2_ref[...]

        r8 = lax.broadcasted_iota(jnp.int32, (HEADS, HID), 0)
        c8 = lax.broadcasted_iota(jnp.int32, (HEADS, HID), 1)
        expand = (c8 // DH == r8).astype(jnp.float32)
        den = jnp.dot(av[:, 64:72], expand, preferred_element_type=jnp.float32) + 1e-16
        att = jnp.concatenate([av[:, :64], bv[:, :64]], axis=1) / den
        o = att + densf
        o_ref[...] = jnp.dot(o, ow_ref[...], preferred_element_type=jnp.float32) + ob_ref[...]

    return pl.pallas_call(
        body,
        out_shape=jax.ShapeDtypeStruct((N, HID), jnp.float32),
    )(accA, accB, nf, dens, x, d_w1, d_b1, d_w2, d_b2, out_w, out_b)


def kernel(x, edge_index, spatial_coords, q_w, k_w, v_w, d_w1, d_b1, d_w2, d_b2,
           dist_emb, dir_emb, sp_w1, sp_b1, sp_w2, sp_b2, temperature, out_w, out_b):
    row = edge_index[0]
    col = edge_index[1]
    row2d = row.reshape(NCHUNK, CH)
    col2d = col.reshape(NCHUNK, CH)
    c128 = jnp.pad(spatial_coords, ((0, 0), (0, HID - 2)))

    w3 = jnp.concatenate([q_w, k_w, v_w], axis=1)
    qkv, qn2, kn2 = _proj_tc(x, w3)
    q = qkv[:, :HID]
    k = qkv[:, HID:2 * HID]
    v = qkv[:, 2 * HID:]

    se_all = jnp.concatenate([
        jnp.repeat(dist_emb, 16, axis=0),
        jnp.tile(dir_emb, (100, 1)),
    ], axis=-1)
    sp_all = (jax.nn.relu(se_all @ sp_w1 + sp_b1) @ sp_w2 + sp_b2)[:, 0]
    spmax = jnp.abs(sp_all).max()
    shift = ((jnp.sqrt(qn2) * jnp.sqrt(kn2)) / math.sqrt(DH) + spmax) \
        / jnp.abs(temperature)[None, :]

    z128 = jnp.zeros((NPAD, HID), jnp.float32)
    qr, kc, vc, dxy = _sc_gather(q, k, v, c128, row2d, col2d)
    nf2 = _sc_nf(x, row2d, col2d, z128)

    dens = _spatial_density(spatial_coords)

    de_pad = jnp.pad(dist_emb, ((0, 28), (0, 0)))
    msgA, msgB = _edge_tc(qr, kc, vc, dxy, de_pad,
                          dir_emb, sp_w1, sp_b1.reshape(1, DH), sp_w2,
                          sp_b2.reshape(1, 1), temperature.reshape(1, HEADS), shift)

    maccA = _sc_scatter(msgA, row2d, z128)
    maccB = _sc_scatter(msgB, row2d, z128)

    accA = maccA[0, :N] + maccA[1, :N]
    accB = maccB[0, :N] + maccB[1, :N]
    nf = nf2[0, :N] + nf2[1, :N]
    return _final_tc(accA, accB, nf, dens.reshape(N, 1), x,
                     d_w1, d_b1.reshape(1, HID // 2), d_w2, d_b2.reshape(1, HID),
                     out_w, out_b.reshape(1, HID))
